# Initial kernel scaffold; baseline (speedup 1.0000x reference)
import jax
import jax.numpy as jnp
from jax.experimental import pallas as pl


def kernel(pos, atomic_number, edge_index, W_in, b_in, W1_l, b1, W1_r, W2_l, b2, W2_r, W_out, b_out):
    raise NotImplementedError("write your pallas kernel here")


# speedup vs baseline: 2.5826x; 2.5826x over previous
import functools

import jax
import jax.numpy as jnp
from jax import lax
from jax.experimental import pallas as pl
from jax.experimental.pallas import tpu as pltpu
from jax.experimental.pallas import tpu_sc as plsc

N = 10000
E = 320000
HID = 128

NC = 2
NS = 16
NW = NC * NS

NP = 10240
RPT = NP // NS
EPAD = 327680
EPW = EPAD // NW
CHUNK = 128
CPW = EPW // CHUNK
PAD_DST = N + 8

_f32 = jnp.float32


def _sc_agg_body(h_hbm, src_hbm, dst_hbm, agg_out, cnt_out,
                 src_v, dst_v, rows_v, ones_v, zcnt_v, agg_sh, cnt_sh, sem):
    c = lax.axis_index("c")
    s = lax.axis_index("s")

    zeros16 = jnp.zeros((16,), _f32)

    def zero_rows(i, carry):
        for j in range(HID // 16):
            rows_v[i, pl.ds(j * 16, 16)] = zeros16
        return carry
    lax.fori_loop(0, CHUNK, zero_rows, 0)

    def zero_cnt(i, carry):
        zcnt_v[pl.ds(i * 16, 16)] = zeros16
        return carry
    lax.fori_loop(0, RPT // 16, zero_cnt, 0)

    for j in range(CHUNK // 16):
        ones_v[pl.ds(j * 16, 16)] = jnp.ones((16,), _f32)

    for r in range(RPT // CHUNK):
        pltpu.sync_copy(rows_v, agg_sh.at[pl.ds(s * RPT + r * CHUNK, CHUNK)])
    pltpu.sync_copy(zcnt_v, cnt_sh.at[pl.ds(s * RPT, RPT)])

    plsc.subcore_barrier()

    base0 = c * (NS * EPW) + s * EPW

    def chunk_body(i, carry):
        base = base0 + i * CHUNK
        pltpu.sync_copy(src_hbm.at[pl.ds(base, CHUNK)], src_v)
        pltpu.sync_copy(dst_hbm.at[pl.ds(base, CHUNK)], dst_v)
        pltpu.async_copy(h_hbm.at[src_v], rows_v, sem).wait()
        pltpu.sync_copy(rows_v, agg_sh.at[dst_v], add=True)
        pltpu.sync_copy(ones_v, cnt_sh.at[dst_v], add=True)
        return carry
    lax.fori_loop(0, CPW, chunk_body, 0)

    plsc.subcore_barrier()

    pltpu.sync_copy(agg_sh.at[pl.ds(s * RPT, RPT)],
                    agg_out.at[c, pl.ds(s * RPT, RPT)])
    pltpu.sync_copy(cnt_sh.at[pl.ds(s * RPT, RPT)],
                    cnt_out.at[c, pl.ds(s * RPT, RPT)])


def _sc_agg(h, src, dst):
    mesh = plsc.VectorSubcoreMesh(core_axis_name="c", subcore_axis_name="s")
    kern = functools.partial(
        pl.kernel,
        out_type=[
            jax.ShapeDtypeStruct((NC, NP, HID), _f32),
            jax.ShapeDtypeStruct((NC, NP), _f32),
        ],
        mesh=mesh,
        scratch_types=[
            pltpu.VMEM((CHUNK,), jnp.int32),
            pltpu.VMEM((CHUNK,), jnp.int32),
            pltpu.VMEM((CHUNK, HID), _f32),
            pltpu.VMEM((CHUNK,), _f32),
            pltpu.VMEM((RPT,), _f32),
            pltpu.VMEM_SHARED((NP, HID), _f32),
            pltpu.VMEM_SHARED((NP,), _f32),
            pltpu.SemaphoreType.DMA,
        ],
    )(_sc_agg_body)
    return kern(h, src, dst)


_ROWS = 1024
_GRID = NP // _ROWS


def _mm(a, w):
    return lax.dot_general(a, w, (((1,), (0,)), ((), ())),
                           precision=lax.Precision.HIGHEST,
                           preferred_element_type=_f32)


def _encode_body(x_ref, w_ref, b_ref, o_ref):
    i = pl.program_id(0)
    h = jnp.maximum(_mm(x_ref[...], w_ref[...]) + b_ref[...], 0.0)
    rows = i * _ROWS + lax.broadcasted_iota(jnp.int32, (_ROWS, 1), 0)
    o_ref[...] = jnp.where(rows < N, h, 0.0)


def _tc_encode(x_pad, w, b):
    return pl.pallas_call(
        _encode_body,
        grid=(_GRID,),
        in_specs=[
            pl.BlockSpec((_ROWS, 8), lambda i: (i, 0)),
            pl.BlockSpec((8, HID), lambda i: (0, 0)),
            pl.BlockSpec((1, HID), lambda i: (0, 0)),
        ],
        out_specs=pl.BlockSpec((_ROWS, HID), lambda i: (i, 0)),
        out_shape=jax.ShapeDtypeStruct((NP, HID), _f32),
    )(x_pad, w, b)


def _sage_body(agg_ref, cnt_ref, h_ref, wl_ref, b_ref, wr_ref, o_ref):
    i = pl.program_id(0)
    cnt = cnt_ref[0] + cnt_ref[1]
    recip = 1.0 / jnp.maximum(cnt, 1.0)
    mean = (agg_ref[0] + agg_ref[1]) * recip
    out = _mm(mean, wl_ref[...]) + b_ref[...] + _mm(h_ref[...], wr_ref[...])
    out = jnp.maximum(out, 0.0)
    rows = i * _ROWS + lax.broadcasted_iota(jnp.int32, (_ROWS, 1), 0)
    o_ref[...] = jnp.where(rows < N, out, 0.0)


def _tc_sage(agg, cnt3, h, wl, b, wr):
    return pl.pallas_call(
        _sage_body,
        grid=(_GRID,),
        in_specs=[
            pl.BlockSpec((NC, _ROWS, HID), lambda i: (0, i, 0)),
            pl.BlockSpec((NC, _ROWS, 1), lambda i: (0, i, 0)),
            pl.BlockSpec((_ROWS, HID), lambda i: (i, 0)),
            pl.BlockSpec((HID, HID), lambda i: (0, 0)),
            pl.BlockSpec((1, HID), lambda i: (0, 0)),
            pl.BlockSpec((HID, HID), lambda i: (0, 0)),
        ],
        out_specs=pl.BlockSpec((_ROWS, HID), lambda i: (i, 0)),
        out_shape=jax.ShapeDtypeStruct((NP, HID), _f32),
    )(agg, cnt3, h, wl, b, wr)


_OROWS = 1000
_OGRID = N // _OROWS


def _out_body(h_ref, w_ref, b_ref, mean_ref, ns_ref):
    i = pl.program_id(0)
    ns = _mm(h_ref[...], w_ref[...]) + b_ref[...]
    ns_ref[...] = ns

    @pl.when(i == 0)
    def _():
        mean_ref[...] = jnp.zeros_like(mean_ref)

    mean_ref[...] += jnp.sum(ns, axis=0, keepdims=True)

    @pl.when(i == _OGRID - 1)
    def _():
        mean_ref[...] = mean_ref[...] * (1.0 / N)


def _tc_out(h, w, b):
    return pl.pallas_call(
        _out_body,
        grid=(_OGRID,),
        in_specs=[
            pl.BlockSpec((_OROWS, HID), lambda i: (i, 0)),
            pl.BlockSpec((HID, HID), lambda i: (0, 0)),
            pl.BlockSpec((1, HID), lambda i: (0, 0)),
        ],
        out_specs=[
            pl.BlockSpec((1, HID), lambda i: (0, 0)),
            pl.BlockSpec((_OROWS, HID), lambda i: (i, 0)),
        ],
        out_shape=[
            jax.ShapeDtypeStruct((1, HID), _f32),
            jax.ShapeDtypeStruct((N, HID), _f32),
        ],
    )(h, w, b)


def kernel(pos, atomic_number, edge_index,
           W_in, b_in, W1_l, b1, W1_r, W2_l, b2, W2_r, W_out, b_out):
    z = atomic_number.astype(_f32)[:, None] / 10.0
    x = jnp.concatenate([z, pos], axis=1)
    x_pad = jnp.pad(x, ((0, NP - N), (0, 4)))
    w_in8 = jnp.pad(W_in, ((0, 0), (0, 4))).T

    src = jnp.concatenate(
        [edge_index[0], jnp.zeros((EPAD - E,), jnp.int32)])
    dst = jnp.concatenate(
        [edge_index[1], jnp.full((EPAD - E,), PAD_DST, jnp.int32)])

    h0 = _tc_encode(x_pad, w_in8, b_in.reshape(1, HID))

    agg1, cnt = _sc_agg(h0, src, dst)
    cnt3 = cnt.reshape(NC, NP, 1)
    h1 = _tc_sage(agg1, cnt3, h0, W1_l.T, b1.reshape(1, HID), W1_r.T)

    agg2, _ = _sc_agg(h1, src, dst)
    h2 = _tc_sage(agg2, cnt3, h1, W2_l.T, b2.reshape(1, HID), W2_r.T)

    mean, node_states = _tc_out(h2, W_out.T, b_out.reshape(1, HID))
    return mean.reshape(HID), node_states


# speedup vs baseline: 4.0055x; 1.5510x over previous
import functools

import jax
import jax.numpy as jnp
from jax import lax
from jax.experimental import pallas as pl
from jax.experimental.pallas import tpu as pltpu
from jax.experimental.pallas import tpu_sc as plsc

N = 10000
E = 320000
HID = 128

NC = 2
NS = 16
NW = NC * NS
NP = 10240
RPT = NP // NS
EPAD = 327680
EPW = EPAD // NW
CHUNK = 128
CPW = EPW // CHUNK
PAD_DST = N + 8

NBUF = 2
NIB = 4

_f32 = jnp.float32


def _sc_agg_body(with_cnt, h_hbm, src_hbm, dst_hbm, *refs):
    if with_cnt:
        (agg_out, cnt_out, sidx, didx, bufs, ones_v, zcnt_v,
         agg_sh, cnt_sh, *sems) = refs
    else:
        (agg_out, sidx, didx, bufs, agg_sh, *sems) = refs
    isems, gsems = sems[:NIB], sems[NIB:]
    c = lax.axis_index("c")
    s = lax.axis_index("s")
    w = c * NS + s

    def idx_start(ci, jj):
        pltpu.async_copy(src_hbm.at[w, ci], sidx.at[jj], isems[jj])
        pltpu.async_copy(dst_hbm.at[w, ci], didx.at[jj], isems[jj])

    def idx_wait(ci, jj):
        pltpu.make_async_copy(src_hbm.at[w, ci], sidx.at[jj], isems[jj]).wait()
        pltpu.make_async_copy(dst_hbm.at[w, ci], didx.at[jj], isems[jj]).wait()

    def gather_start(jj, j):
        pltpu.async_copy(h_hbm.at[sidx.at[jj]], bufs.at[j], gsems[j])

    def gather_wait(jj, j):
        pltpu.make_async_copy(h_hbm.at[sidx.at[jj]], bufs.at[j], gsems[j]).wait()

    for jj in range(NIB):
        idx_start(jj, jj)

    zeros16 = jnp.zeros((16,), _f32)

    def zero_rows(i, carry):
        for j in range(HID // 16):
            bufs[0, i, pl.ds(j * 16, 16)] = zeros16
        return carry
    lax.fori_loop(0, CHUNK, zero_rows, 0)
    zrow = bufs.at[0]
    for r in range(RPT // CHUNK):
        pltpu.sync_copy(zrow, agg_sh.at[pl.ds(s * RPT + r * CHUNK, CHUNK)])

    if with_cnt:
        def zero_cnt(i, carry):
            zcnt_v[pl.ds(i * 16, 16)] = zeros16
            return carry
        lax.fori_loop(0, RPT // 16, zero_cnt, 0)
        for j in range(CHUNK // 16):
            ones_v[pl.ds(j * 16, 16)] = jnp.ones((16,), _f32)
        pltpu.sync_copy(zcnt_v, cnt_sh.at[pl.ds(s * RPT, RPT)])

    for j in range(NBUF):
        idx_wait(j, j)
        gather_start(j, j)

    plsc.subcore_barrier()

    def do_chunk(ci, j, jj, jjn, start_gather, start_idx):
        gather_wait(jj, j)
        pltpu.sync_copy(bufs.at[j], agg_sh.at[didx.at[jj]], add=True)
        if with_cnt:
            pltpu.sync_copy(ones_v, cnt_sh.at[didx.at[jj]], add=True)
        if start_gather:
            idx_wait(ci + NBUF, jjn)
            gather_start(jjn, j)
        if start_idx:
            idx_start(ci + NIB, jj)

    GM = (CPW - NIB) // NIB

    def group_body(g, carry):
        base = g * NIB
        for u in range(NIB):
            ci = base + u
            do_chunk(ci, u % NBUF, u, (u + NBUF) % NIB, True, True)
        return carry
    lax.fori_loop(0, GM, group_body, 0)

    base = CPW - NIB
    for u in range(NIB):
        ci = base + u
        do_chunk(ci, u % NBUF, u, (u + NBUF) % NIB, u < NIB - NBUF, False)

    plsc.subcore_barrier()

    pltpu.sync_copy(agg_sh.at[pl.ds(s * RPT, RPT)],
                    agg_out.at[c, pl.ds(s * RPT, RPT)])
    if with_cnt:
        pltpu.sync_copy(cnt_sh.at[pl.ds(s * RPT, RPT)],
                        cnt_out.at[c, pl.ds(s * RPT, RPT)])


def _sc_agg(h, src3, dst3, with_cnt):
    mesh = plsc.VectorSubcoreMesh(core_axis_name="c", subcore_axis_name="s")
    out_type = [jax.ShapeDtypeStruct((NC, NP, HID), _f32)]
    scratch = [
        pltpu.VMEM((NIB, CHUNK), jnp.int32),
        pltpu.VMEM((NIB, CHUNK), jnp.int32),
        pltpu.VMEM((NBUF, CHUNK, HID), _f32),
    ]
    if with_cnt:
        out_type.append(jax.ShapeDtypeStruct((NC, NP), _f32))
        scratch += [pltpu.VMEM((CHUNK,), _f32), pltpu.VMEM((RPT,), _f32)]
    scratch.append(pltpu.VMEM_SHARED((NP, HID), _f32))
    if with_cnt:
        scratch.append(pltpu.VMEM_SHARED((NP,), _f32))
    scratch += [pltpu.SemaphoreType.DMA] * (NIB + NBUF)
    kern = functools.partial(
        pl.kernel,
        out_type=out_type,
        mesh=mesh,
        scratch_types=scratch,
    )(functools.partial(_sc_agg_body, with_cnt))
    return kern(h, src3, dst3)


_ROWS = 1024
_GRID = NP // _ROWS


def _mm(a, w):
    return lax.dot_general(a, w, (((1,), (0,)), ((), ())),
                           precision=lax.Precision.HIGHEST,
                           preferred_element_type=_f32)


def _encode_body(x_ref, w_ref, b_ref, o_ref):
    i = pl.program_id(0)
    h = jnp.maximum(_mm(x_ref[...], w_ref[...]) + b_ref[...], 0.0)
    rows = i * _ROWS + lax.broadcasted_iota(jnp.int32, (_ROWS, 1), 0)
    o_ref[...] = jnp.where(rows < N, h, 0.0)


def _tc_encode(x_pad, w, b):
    return pl.pallas_call(
        _encode_body,
        grid=(_GRID,),
        in_specs=[
            pl.BlockSpec((_ROWS, 8), lambda i: (i, 0)),
            pl.BlockSpec((8, HID), lambda i: (0, 0)),
            pl.BlockSpec((1, HID), lambda i: (0, 0)),
        ],
        out_specs=pl.BlockSpec((_ROWS, HID), lambda i: (i, 0)),
        out_shape=jax.ShapeDtypeStruct((NP, HID), _f32),
    )(x_pad, w, b)


def _sage_body(agg_ref, cnt_ref, h_ref, wl_ref, b_ref, wr_ref, o_ref):
    i = pl.program_id(0)
    cnt = cnt_ref[0] + cnt_ref[1]
    recip = 1.0 / jnp.maximum(cnt, 1.0)
    mean = (agg_ref[0] + agg_ref[1]) * recip
    out = _mm(mean, wl_ref[...]) + b_ref[...] + _mm(h_ref[...], wr_ref[...])
    out = jnp.maximum(out, 0.0)
    rows = i * _ROWS + lax.broadcasted_iota(jnp.int32, (_ROWS, 1), 0)
    o_ref[...] = jnp.where(rows < N, out, 0.0)


def _tc_sage(agg, cnt3, h, wl, b, wr):
    return pl.pallas_call(
        _sage_body,
        grid=(_GRID,),
        in_specs=[
            pl.BlockSpec((NC, _ROWS, HID), lambda i: (0, i, 0)),
            pl.BlockSpec((NC, _ROWS, 1), lambda i: (0, i, 0)),
            pl.BlockSpec((_ROWS, HID), lambda i: (i, 0)),
            pl.BlockSpec((HID, HID), lambda i: (0, 0)),
            pl.BlockSpec((1, HID), lambda i: (0, 0)),
            pl.BlockSpec((HID, HID), lambda i: (0, 0)),
        ],
        out_specs=pl.BlockSpec((_ROWS, HID), lambda i: (i, 0)),
        out_shape=jax.ShapeDtypeStruct((NP, HID), _f32),
    )(agg, cnt3, h, wl, b, wr)


_OROWS = 1000
_OGRID = N // _OROWS


def _out_body(h_ref, w_ref, b_ref, mean_ref, ns_ref):
    i = pl.program_id(0)
    ns = _mm(h_ref[...], w_ref[...]) + b_ref[...]
    ns_ref[...] = ns

    @pl.when(i == 0)
    def _():
        mean_ref[...] = jnp.zeros_like(mean_ref)

    mean_ref[...] += jnp.sum(ns, axis=0, keepdims=True)

    @pl.when(i == _OGRID - 1)
    def _():
        mean_ref[...] = mean_ref[...] * (1.0 / N)


def _tc_out(h, w, b):
    return pl.pallas_call(
        _out_body,
        grid=(_OGRID,),
        in_specs=[
            pl.BlockSpec((_OROWS, HID), lambda i: (i, 0)),
            pl.BlockSpec((HID, HID), lambda i: (0, 0)),
            pl.BlockSpec((1, HID), lambda i: (0, 0)),
        ],
        out_specs=[
            pl.BlockSpec((1, HID), lambda i: (0, 0)),
            pl.BlockSpec((_OROWS, HID), lambda i: (i, 0)),
        ],
        out_shape=[
            jax.ShapeDtypeStruct((1, HID), _f32),
            jax.ShapeDtypeStruct((N, HID), _f32),
        ],
    )(h, w, b)


def kernel(pos, atomic_number, edge_index,
           W_in, b_in, W1_l, b1, W1_r, W2_l, b2, W2_r, W_out, b_out):
    z = atomic_number.astype(_f32)[:, None] / 10.0
    x = jnp.concatenate([z, pos], axis=1)
    x_pad = jnp.pad(x, ((0, NP - N), (0, 4)))
    w_in8 = jnp.pad(W_in, ((0, 0), (0, 4))).T

    src3 = jnp.concatenate(
        [edge_index[0], jnp.zeros((EPAD - E,), jnp.int32)]
    ).reshape(NW, CPW, CHUNK)
    dst3 = jnp.concatenate(
        [edge_index[1], jnp.full((EPAD - E,), PAD_DST, jnp.int32)]
    ).reshape(NW, CPW, CHUNK)

    h0 = _tc_encode(x_pad, w_in8, b_in.reshape(1, HID))

    agg1, cnt = _sc_agg(h0, src3, dst3, True)
    cnt3 = cnt.reshape(NC, NP, 1)
    h1 = _tc_sage(agg1, cnt3, h0, W1_l.T, b1.reshape(1, HID), W1_r.T)

    (agg2,) = _sc_agg(h1, src3, dst3, False)
    h2 = _tc_sage(agg2, cnt3, h1, W2_l.T, b2.reshape(1, HID), W2_r.T)

    mean, node_states = _tc_out(h2, W_out.T, b_out.reshape(1, HID))
    return mean.reshape(HID), node_states


# speedup vs baseline: 4.1978x; 1.0480x over previous
import functools

import jax
import jax.numpy as jnp
from jax import lax
from jax.experimental import pallas as pl
from jax.experimental.pallas import tpu as pltpu
from jax.experimental.pallas import tpu_sc as plsc

N = 10000
E = 320000
HID = 128

NC = 2
NS = 16
NW = NC * NS
NP = 10240
RPT = NP // NS
CHUNK = 128
CPW0 = 128
CPW1 = 32
EPAD = NS * (CPW0 + CPW1) * CHUNK
PAD_DST = N + 8

NBUF = 2
NIB = 4

_f32 = jnp.float32


def _sc_agg_body(with_cnt, h_hbm, src_hbm, dst_hbm, *refs):
    if with_cnt:
        (agg_out, cnt_out, sidx, didx, bufs, ones_v, zcnt_v,
         agg_sh, cnt_sh, *sems) = refs
    else:
        (agg_out, sidx, didx, bufs, agg_sh, *sems) = refs
    isems, gsems = sems[:NIB], sems[NIB:]
    c = lax.axis_index("c")
    s = lax.axis_index("s")
    cpw = jnp.where(c == 0, CPW0, CPW1)
    chunk0 = jnp.where(c == 0, s * CPW0, NS * CPW0 + s * CPW1)

    def idx_start(ci, jj):
        e0 = (chunk0 + ci) * CHUNK
        pltpu.async_copy(src_hbm.at[pl.ds(e0, CHUNK)], sidx.at[jj], isems[jj])
        pltpu.async_copy(dst_hbm.at[pl.ds(e0, CHUNK)], didx.at[jj], isems[jj])

    def idx_wait(ci, jj):
        e0 = (chunk0 + ci) * CHUNK
        pltpu.make_async_copy(
            src_hbm.at[pl.ds(e0, CHUNK)], sidx.at[jj], isems[jj]).wait()
        pltpu.make_async_copy(
            dst_hbm.at[pl.ds(e0, CHUNK)], didx.at[jj], isems[jj]).wait()

    def gather_start(jj, j):
        pltpu.async_copy(h_hbm.at[sidx.at[jj]], bufs.at[j], gsems[j])

    def gather_wait(jj, j):
        pltpu.make_async_copy(h_hbm.at[sidx.at[jj]], bufs.at[j], gsems[j]).wait()

    for jj in range(NIB):
        idx_start(jj, jj)

    zeros16 = jnp.zeros((16,), _f32)

    def zero_rows(i, carry):
        for j in range(HID // 16):
            bufs[0, i, pl.ds(j * 16, 16)] = zeros16
        return carry
    lax.fori_loop(0, CHUNK, zero_rows, 0)
    zrow = bufs.at[0]
    for r in range(RPT // CHUNK):
        pltpu.sync_copy(zrow, agg_sh.at[pl.ds(s * RPT + r * CHUNK, CHUNK)])

    if with_cnt:
        def zero_cnt(i, carry):
            zcnt_v[pl.ds(i * 16, 16)] = zeros16
            return carry
        lax.fori_loop(0, RPT // 16, zero_cnt, 0)
        for j in range(CHUNK // 16):
            ones_v[pl.ds(j * 16, 16)] = jnp.ones((16,), _f32)
        pltpu.sync_copy(zcnt_v, cnt_sh.at[pl.ds(s * RPT, RPT)])

    for j in range(NBUF):
        idx_wait(j, j)
        gather_start(j, j)

    plsc.subcore_barrier()

    def do_chunk(ci, j, jj, jjn, start_gather, start_idx):
        gather_wait(jj, j)
        pltpu.sync_copy(bufs.at[j], agg_sh.at[didx.at[jj]], add=True)
        if with_cnt:
            pltpu.sync_copy(ones_v, cnt_sh.at[didx.at[jj]], add=True)
        if start_gather:
            idx_wait(ci + NBUF, jjn)
            gather_start(jjn, j)
        if start_idx:
            idx_start(ci + NIB, jj)

    gm = cpw // NIB - 1

    def group_body(g, carry):
        base = g * NIB
        for u in range(NIB):
            ci = base + u
            do_chunk(ci, u % NBUF, u, (u + NBUF) % NIB, True, True)
        return carry
    lax.fori_loop(0, gm, group_body, 0)

    base = cpw - NIB
    for u in range(NIB):
        ci = base + u
        do_chunk(ci, u % NBUF, u, (u + NBUF) % NIB, u < NIB - NBUF, False)

    plsc.subcore_barrier()

    pltpu.sync_copy(agg_sh.at[pl.ds(s * RPT, RPT)],
                    agg_out.at[c, pl.ds(s * RPT, RPT)])
    if with_cnt:
        pltpu.sync_copy(cnt_sh.at[pl.ds(s * RPT, RPT)],
                        cnt_out.at[c, pl.ds(s * RPT, RPT)])


def _sc_agg(h, src, dst, with_cnt):
    mesh = plsc.VectorSubcoreMesh(core_axis_name="c", subcore_axis_name="s")
    out_type = [jax.ShapeDtypeStruct((NC, NP, HID), _f32)]
    scratch = [
        pltpu.VMEM((NIB, CHUNK), jnp.int32),
        pltpu.VMEM((NIB, CHUNK), jnp.int32),
        pltpu.VMEM((NBUF, CHUNK, HID), _f32),
    ]
    if with_cnt:
        out_type.append(jax.ShapeDtypeStruct((NC, NP), _f32))
        scratch += [pltpu.VMEM((CHUNK,), _f32), pltpu.VMEM((RPT,), _f32)]
    scratch.append(pltpu.VMEM_SHARED((NP, HID), _f32))
    if with_cnt:
        scratch.append(pltpu.VMEM_SHARED((NP,), _f32))
    scratch += [pltpu.SemaphoreType.DMA] * (NIB + NBUF)
    kern = functools.partial(
        pl.kernel,
        out_type=out_type,
        mesh=mesh,
        scratch_types=scratch,
    )(functools.partial(_sc_agg_body, with_cnt))
    return kern(h, src, dst)


_ROWS = 1024
_GRID = NP // _ROWS


def _mm(a, w):
    return lax.dot_general(a, w, (((1,), (0,)), ((), ())),
                           precision=lax.Precision.HIGHEST,
                           preferred_element_type=_f32)


def _encode_body(x_ref, w_ref, b_ref, o_ref):
    i = pl.program_id(0)
    h = jnp.maximum(_mm(x_ref[...], w_ref[...]) + b_ref[...], 0.0)
    rows = i * _ROWS + lax.broadcasted_iota(jnp.int32, (_ROWS, 1), 0)
    o_ref[...] = jnp.where(rows < N, h, 0.0)


def _tc_encode(x_pad, w, b):
    return pl.pallas_call(
        _encode_body,
        grid=(_GRID,),
        in_specs=[
            pl.BlockSpec((_ROWS, 8), lambda i: (i, 0)),
            pl.BlockSpec((8, HID), lambda i: (0, 0)),
            pl.BlockSpec((1, HID), lambda i: (0, 0)),
        ],
        out_specs=pl.BlockSpec((_ROWS, HID), lambda i: (i, 0)),
        out_shape=jax.ShapeDtypeStruct((NP, HID), _f32),
    )(x_pad, w, b)


def _sage_body(agg_ref, cnt_ref, h_ref, wl_ref, b_ref, wr_ref, o_ref):
    i = pl.program_id(0)
    cnt = cnt_ref[0] + cnt_ref[1]
    recip = 1.0 / jnp.maximum(cnt, 1.0)
    mean = (agg_ref[0] + agg_ref[1]) * recip
    out = _mm(mean, wl_ref[...]) + b_ref[...] + _mm(h_ref[...], wr_ref[...])
    out = jnp.maximum(out, 0.0)
    rows = i * _ROWS + lax.broadcasted_iota(jnp.int32, (_ROWS, 1), 0)
    o_ref[...] = jnp.where(rows < N, out, 0.0)


def _tc_sage(agg, cnt3, h, wl, b, wr):
    return pl.pallas_call(
        _sage_body,
        grid=(_GRID,),
        in_specs=[
            pl.BlockSpec((NC, _ROWS, HID), lambda i: (0, i, 0)),
            pl.BlockSpec((NC, _ROWS, 1), lambda i: (0, i, 0)),
            pl.BlockSpec((_ROWS, HID), lambda i: (i, 0)),
            pl.BlockSpec((HID, HID), lambda i: (0, 0)),
            pl.BlockSpec((1, HID), lambda i: (0, 0)),
            pl.BlockSpec((HID, HID), lambda i: (0, 0)),
        ],
        out_specs=pl.BlockSpec((_ROWS, HID), lambda i: (i, 0)),
        out_shape=jax.ShapeDtypeStruct((NP, HID), _f32),
    )(agg, cnt3, h, wl, b, wr)


_OROWS = 1000
_OGRID = N // _OROWS


def _out_body(h_ref, w_ref, b_ref, mean_ref, ns_ref):
    i = pl.program_id(0)
    ns = _mm(h_ref[...], w_ref[...]) + b_ref[...]
    ns_ref[...] = ns

    @pl.when(i == 0)
    def _():
        mean_ref[...] = jnp.zeros_like(mean_ref)

    mean_ref[...] += jnp.sum(ns, axis=0, keepdims=True)

    @pl.when(i == _OGRID - 1)
    def _():
        mean_ref[...] = mean_ref[...] * (1.0 / N)


def _tc_out(h, w, b):
    return pl.pallas_call(
        _out_body,
        grid=(_OGRID,),
        in_specs=[
            pl.BlockSpec((_OROWS, HID), lambda i: (i, 0)),
            pl.BlockSpec((HID, HID), lambda i: (0, 0)),
            pl.BlockSpec((1, HID), lambda i: (0, 0)),
        ],
        out_specs=[
            pl.BlockSpec((1, HID), lambda i: (0, 0)),
            pl.BlockSpec((_OROWS, HID), lambda i: (i, 0)),
        ],
        out_shape=[
            jax.ShapeDtypeStruct((1, HID), _f32),
            jax.ShapeDtypeStruct((N, HID), _f32),
        ],
    )(h, w, b)


def kernel(pos, atomic_number, edge_index,
           W_in, b_in, W1_l, b1, W1_r, W2_l, b2, W2_r, W_out, b_out):
    z = atomic_number.astype(_f32)[:, None] / 10.0
    x = jnp.concatenate([z, pos], axis=1)
    x_pad = jnp.pad(x, ((0, NP - N), (0, 4)))
    w_in8 = jnp.pad(W_in, ((0, 0), (0, 4))).T

    src = jnp.concatenate(
        [edge_index[0], jnp.zeros((EPAD - E,), jnp.int32)])
    dst = jnp.concatenate(
        [edge_index[1], jnp.full((EPAD - E,), PAD_DST, jnp.int32)])

    h0 = _tc_encode(x_pad, w_in8, b_in.reshape(1, HID))

    agg1, cnt = _sc_agg(h0, src, dst, True)
    cnt3 = cnt.reshape(NC, NP, 1)
    h1 = _tc_sage(agg1, cnt3, h0, W1_l.T, b1.reshape(1, HID), W1_r.T)

    (agg2,) = _sc_agg(h1, src, dst, False)
    h2 = _tc_sage(agg2, cnt3, h1, W2_l.T, b2.reshape(1, HID), W2_r.T)

    mean, node_states = _tc_out(h2, W_out.T, b_out.reshape(1, HID))
    return mean.reshape(HID), node_states


# speedup vs baseline: 4.2020x; 1.0010x over previous
import functools

import jax
import jax.numpy as jnp
from jax import lax
from jax.experimental import pallas as pl
from jax.experimental.pallas import tpu as pltpu
from jax.experimental.pallas import tpu_sc as plsc

N = 10000
E = 320000
HID = 128

NC = 2
NS = 16
NW = NC * NS
NP = 10240
RPT = NP // NS
CHUNK = 128
CPW0 = 128
CPW1 = 32
EPAD = NS * (CPW0 + CPW1) * CHUNK
PAD_DST = N + 8

NBUF = 2
NIB = 4

_f32 = jnp.float32


def _sc_agg_body(with_cnt, h_hbm, src_hbm, dst_hbm, *refs):
    if with_cnt:
        (agg_out, cnt_out, sidx, didx, bufs, ones_v, zcnt_v,
         agg_sh, cnt_sh, *sems) = refs
    else:
        (agg_out, sidx, didx, bufs, agg_sh, *sems) = refs
    isems, gsems = sems[:NIB], sems[NIB:]
    c = lax.axis_index("c")
    s = lax.axis_index("s")
    cpw = jnp.where(c == 0, CPW0, CPW1)
    chunk0 = jnp.where(c == 0, s * CPW0, NS * CPW0 + s * CPW1)

    def idx_start(ci, jj):
        e0 = (chunk0 + ci) * CHUNK
        pltpu.async_copy(src_hbm.at[pl.ds(e0, CHUNK)], sidx.at[jj], isems[jj])
        pltpu.async_copy(dst_hbm.at[pl.ds(e0, CHUNK)], didx.at[jj], isems[jj])

    def idx_wait(ci, jj):
        e0 = (chunk0 + ci) * CHUNK
        pltpu.make_async_copy(
            src_hbm.at[pl.ds(e0, CHUNK)], sidx.at[jj], isems[jj]).wait()
        pltpu.make_async_copy(
            dst_hbm.at[pl.ds(e0, CHUNK)], didx.at[jj], isems[jj]).wait()

    def gather_start(jj, j):
        pltpu.async_copy(h_hbm.at[sidx.at[jj]], bufs.at[j], gsems[j])

    def gather_wait(jj, j):
        pltpu.make_async_copy(h_hbm.at[sidx.at[jj]], bufs.at[j], gsems[j]).wait()

    with jax.named_scope("sc_prologue"):
        for jj in range(NIB):
            idx_start(jj, jj)

    zeros16 = jnp.zeros((16,), _f32)
    _zs = jax.named_scope("sc_zero_init")
    _zs.__enter__()

    def zero_rows(i, carry):
        for j in range(HID // 16):
            bufs[0, i, pl.ds(j * 16, 16)] = zeros16
        return carry
    lax.fori_loop(0, CHUNK, zero_rows, 0)
    zrow = bufs.at[0]
    for r in range(RPT // CHUNK):
        pltpu.sync_copy(zrow, agg_sh.at[pl.ds(s * RPT + r * CHUNK, CHUNK)])

    if with_cnt:
        def zero_cnt(i, carry):
            zcnt_v[pl.ds(i * 16, 16)] = zeros16
            return carry
        lax.fori_loop(0, RPT // 16, zero_cnt, 0)
        for j in range(CHUNK // 16):
            ones_v[pl.ds(j * 16, 16)] = jnp.ones((16,), _f32)
        pltpu.sync_copy(zcnt_v, cnt_sh.at[pl.ds(s * RPT, RPT)])

    _zs.__exit__(None, None, None)

    with jax.named_scope("sc_prime"):
        for j in range(NBUF):
            idx_wait(j, j)
            gather_start(j, j)

        plsc.subcore_barrier()

    def do_chunk(ci, j, jj, jjn, start_gather, start_idx):
        gather_wait(jj, j)
        pltpu.sync_copy(bufs.at[j], agg_sh.at[didx.at[jj]], add=True)
        if with_cnt:
            pltpu.sync_copy(ones_v, cnt_sh.at[didx.at[jj]], add=True)
        if start_gather:
            idx_wait(ci + NBUF, jjn)
            gather_start(jjn, j)
        if start_idx:
            idx_start(ci + NIB, jj)

    gm = cpw // NIB - 1

    def group_body(g, carry):
        base = g * NIB
        for u in range(NIB):
            ci = base + u
            do_chunk(ci, u % NBUF, u, (u + NBUF) % NIB, True, True)
        return carry
    with jax.named_scope("sc_mainloop"):
        lax.fori_loop(0, gm, group_body, 0)

    base = cpw - NIB
    for u in range(NIB):
        ci = base + u
        do_chunk(ci, u % NBUF, u, (u + NBUF) % NIB, u < NIB - NBUF, False)

    with jax.named_scope("sc_tail_barrier"):
        plsc.subcore_barrier()

    with jax.named_scope("sc_writeout"):
        pltpu.sync_copy(agg_sh.at[pl.ds(s * RPT, RPT)],
                        agg_out.at[c, pl.ds(s * RPT, RPT)])
        if with_cnt:
            pltpu.sync_copy(cnt_sh.at[pl.ds(s * RPT, RPT)],
                            cnt_out.at[c, pl.ds(s * RPT, RPT)])


def _sc_agg(h, src, dst, with_cnt):
    mesh = plsc.VectorSubcoreMesh(core_axis_name="c", subcore_axis_name="s")
    out_type = [jax.ShapeDtypeStruct((NC, NP, HID), _f32)]
    scratch = [
        pltpu.VMEM((NIB, CHUNK), jnp.int32),
        pltpu.VMEM((NIB, CHUNK), jnp.int32),
        pltpu.VMEM((NBUF, CHUNK, HID), _f32),
    ]
    if with_cnt:
        out_type.append(jax.ShapeDtypeStruct((NC, NP), _f32))
        scratch += [pltpu.VMEM((CHUNK,), _f32), pltpu.VMEM((RPT,), _f32)]
    scratch.append(pltpu.VMEM_SHARED((NP, HID), _f32))
    if with_cnt:
        scratch.append(pltpu.VMEM_SHARED((NP,), _f32))
    scratch += [pltpu.SemaphoreType.DMA] * (NIB + NBUF)
    kern = functools.partial(
        pl.kernel,
        out_type=out_type,
        mesh=mesh,
        scratch_types=scratch,
    )(functools.partial(_sc_agg_body, with_cnt))
    return kern(h, src, dst)


_ROWS = 1024
_GRID = NP // _ROWS


def _mm(a, w):
    return lax.dot_general(a, w, (((1,), (0,)), ((), ())),
                           precision=lax.Precision.HIGHEST,
                           preferred_element_type=_f32)


def _encode_body(x_ref, w_ref, b_ref, o_ref):
    i = pl.program_id(0)
    h = jnp.maximum(_mm(x_ref[...], w_ref[...]) + b_ref[...], 0.0)
    rows = i * _ROWS + lax.broadcasted_iota(jnp.int32, (_ROWS, 1), 0)
    o_ref[...] = jnp.where(rows < N, h, 0.0)


def _tc_encode(x_pad, w, b):
    return pl.pallas_call(
        _encode_body,
        grid=(_GRID,),
        in_specs=[
            pl.BlockSpec((_ROWS, 8), lambda i: (i, 0)),
            pl.BlockSpec((8, HID), lambda i: (0, 0)),
            pl.BlockSpec((1, HID), lambda i: (0, 0)),
        ],
        out_specs=pl.BlockSpec((_ROWS, HID), lambda i: (i, 0)),
        out_shape=jax.ShapeDtypeStruct((NP, HID), _f32),
    )(x_pad, w, b)


def _sage_body(agg_ref, cnt_ref, h_ref, wl_ref, b_ref, wr_ref, o_ref):
    i = pl.program_id(0)
    cnt = cnt_ref[0] + cnt_ref[1]
    recip = 1.0 / jnp.maximum(cnt, 1.0)
    mean = (agg_ref[0] + agg_ref[1]) * recip
    out = _mm(mean, wl_ref[...]) + b_ref[...] + _mm(h_ref[...], wr_ref[...])
    out = jnp.maximum(out, 0.0)
    rows = i * _ROWS + lax.broadcasted_iota(jnp.int32, (_ROWS, 1), 0)
    o_ref[...] = jnp.where(rows < N, out, 0.0)


def _tc_sage(agg, cnt3, h, wl, b, wr):
    return pl.pallas_call(
        _sage_body,
        grid=(_GRID,),
        in_specs=[
            pl.BlockSpec((NC, _ROWS, HID), lambda i: (0, i, 0)),
            pl.BlockSpec((NC, _ROWS, 1), lambda i: (0, i, 0)),
            pl.BlockSpec((_ROWS, HID), lambda i: (i, 0)),
            pl.BlockSpec((HID, HID), lambda i: (0, 0)),
            pl.BlockSpec((1, HID), lambda i: (0, 0)),
            pl.BlockSpec((HID, HID), lambda i: (0, 0)),
        ],
        out_specs=pl.BlockSpec((_ROWS, HID), lambda i: (i, 0)),
        out_shape=jax.ShapeDtypeStruct((NP, HID), _f32),
    )(agg, cnt3, h, wl, b, wr)


_OROWS = 1000
_OGRID = N // _OROWS


def _out_body(h_ref, w_ref, b_ref, mean_ref, ns_ref):
    i = pl.program_id(0)
    ns = _mm(h_ref[...], w_ref[...]) + b_ref[...]
    ns_ref[...] = ns

    @pl.when(i == 0)
    def _():
        mean_ref[...] = jnp.zeros_like(mean_ref)

    mean_ref[...] += jnp.sum(ns, axis=0, keepdims=True)

    @pl.when(i == _OGRID - 1)
    def _():
        mean_ref[...] = mean_ref[...] * (1.0 / N)


def _tc_out(h, w, b):
    return pl.pallas_call(
        _out_body,
        grid=(_OGRID,),
        in_specs=[
            pl.BlockSpec((_OROWS, HID), lambda i: (i, 0)),
            pl.BlockSpec((HID, HID), lambda i: (0, 0)),
            pl.BlockSpec((1, HID), lambda i: (0, 0)),
        ],
        out_specs=[
            pl.BlockSpec((1, HID), lambda i: (0, 0)),
            pl.BlockSpec((_OROWS, HID), lambda i: (i, 0)),
        ],
        out_shape=[
            jax.ShapeDtypeStruct((1, HID), _f32),
            jax.ShapeDtypeStruct((N, HID), _f32),
        ],
    )(h, w, b)


def kernel(pos, atomic_number, edge_index,
           W_in, b_in, W1_l, b1, W1_r, W2_l, b2, W2_r, W_out, b_out):
    z = atomic_number.astype(_f32)[:, None] / 10.0
    x = jnp.concatenate([z, pos], axis=1)
    x_pad = jnp.pad(x, ((0, NP - N), (0, 4)))
    w_in8 = jnp.pad(W_in, ((0, 0), (0, 4))).T

    src = jnp.concatenate(
        [edge_index[0], jnp.zeros((EPAD - E,), jnp.int32)])
    dst = jnp.concatenate(
        [edge_index[1], jnp.full((EPAD - E,), PAD_DST, jnp.int32)])

    h0 = _tc_encode(x_pad, w_in8, b_in.reshape(1, HID))

    agg1, cnt = _sc_agg(h0, src, dst, True)
    cnt3 = cnt.reshape(NC, NP, 1)
    h1 = _tc_sage(agg1, cnt3, h0, W1_l.T, b1.reshape(1, HID), W1_r.T)

    (agg2,) = _sc_agg(h1, src, dst, False)
    h2 = _tc_sage(agg2, cnt3, h1, W2_l.T, b2.reshape(1, HID), W2_r.T)

    mean, node_states = _tc_out(h2, W_out.T, b_out.reshape(1, HID))
    return mean.reshape(HID), node_states


# speedup vs baseline: 12.0935x; 2.8780x over previous
import functools

import jax
import jax.numpy as jnp
from jax import lax
from jax.experimental import pallas as pl
from jax.experimental.pallas import tpu as pltpu
from jax.experimental.pallas import tpu_sc as plsc

N = 10000
E = 320000
HID = 128

NC = 2
NS = 16
NW = NC * NS
NP = 10240
RPT = NP // NS
CHUNK = 128
CPW0 = 80
CPW1 = 80
EPAD = NS * (CPW0 + CPW1) * CHUNK

NBUF = 2
NIB = 4

_f32 = jnp.float32


def _sc_agg_body(with_cnt, h_hbm, src_hbm, dst_hbm, *refs):
    if with_cnt:
        (agg_out, cnt_out, sidx, didx, bufs, ones_v, zcnt_v,
         agg_sh, cnt_sh, *sems) = refs
    else:
        (agg_out, sidx, didx, bufs, agg_sh, *sems) = refs
    isems, gsems = sems[:NIB], sems[NIB:]
    c = lax.axis_index("c")
    s = lax.axis_index("s")
    cpw = jnp.where(c == 0, CPW0, CPW1)
    chunk0 = jnp.where(c == 0, s * CPW0, NS * CPW0 + s * CPW1)

    def idx_start(ci, jj):
        e0 = (chunk0 + ci) * CHUNK
        pltpu.async_copy(src_hbm.at[pl.ds(e0, CHUNK)], sidx.at[jj], isems[jj])
        pltpu.async_copy(dst_hbm.at[pl.ds(e0, CHUNK)], didx.at[jj], isems[jj])

    def idx_wait(ci, jj):
        e0 = (chunk0 + ci) * CHUNK
        pltpu.make_async_copy(
            src_hbm.at[pl.ds(e0, CHUNK)], sidx.at[jj], isems[jj]).wait()
        pltpu.make_async_copy(
            dst_hbm.at[pl.ds(e0, CHUNK)], didx.at[jj], isems[jj]).wait()

    def gather_start(jj, j):
        pltpu.async_copy(h_hbm.at[sidx.at[jj]], bufs.at[j], gsems[j])

    def gather_wait(jj, j):
        pltpu.make_async_copy(h_hbm.at[sidx.at[jj]], bufs.at[j], gsems[j]).wait()

    with jax.named_scope("sc_prologue"):
        for jj in range(NIB):
            idx_start(jj, jj)

    zeros16 = jnp.zeros((16,), _f32)
    _zs = jax.named_scope("sc_zero_init")
    _zs.__enter__()

    def zero_rows(i, carry):
        for j in range(HID // 16):
            bufs[0, i, pl.ds(j * 16, 16)] = zeros16
        return carry
    lax.fori_loop(0, CHUNK, zero_rows, 0)
    zrow = bufs.at[0]
    for r in range(RPT // CHUNK):
        pltpu.sync_copy(zrow, agg_sh.at[pl.ds(s * RPT + r * CHUNK, CHUNK)])

    if with_cnt:
        def zero_cnt(i, carry):
            zcnt_v[pl.ds(i * 16, 16)] = zeros16
            return carry
        lax.fori_loop(0, RPT // 16, zero_cnt, 0)
        for j in range(CHUNK // 16):
            ones_v[pl.ds(j * 16, 16)] = jnp.ones((16,), _f32)
        pltpu.sync_copy(zcnt_v, cnt_sh.at[pl.ds(s * RPT, RPT)])

    _zs.__exit__(None, None, None)

    with jax.named_scope("sc_prime"):
        for j in range(NBUF):
            idx_wait(j, j)
            gather_start(j, j)

        plsc.subcore_barrier()

    def do_chunk(ci, j, jj, jjn, start_gather, start_idx):
        gather_wait(jj, j)
        pltpu.sync_copy(bufs.at[j], agg_sh.at[didx.at[jj]], add=True)
        if with_cnt:
            pltpu.sync_copy(ones_v, cnt_sh.at[didx.at[jj]], add=True)
        if start_gather:
            idx_wait(ci + NBUF, jjn)
            gather_start(jjn, j)
        if start_idx:
            idx_start(ci + NIB, jj)

    gm = cpw // NIB - 1

    def group_body(g, carry):
        base = g * NIB
        for u in range(NIB):
            ci = base + u
            do_chunk(ci, u % NBUF, u, (u + NBUF) % NIB, True, True)
        return carry
    with jax.named_scope("sc_mainloop"):
        lax.fori_loop(0, gm, group_body, 0)

    base = cpw - NIB
    for u in range(NIB):
        ci = base + u
        do_chunk(ci, u % NBUF, u, (u + NBUF) % NIB, u < NIB - NBUF, False)

    with jax.named_scope("sc_tail_barrier"):
        plsc.subcore_barrier()

    with jax.named_scope("sc_writeout"):
        pltpu.sync_copy(agg_sh.at[pl.ds(s * RPT, RPT)],
                        agg_out.at[c, pl.ds(s * RPT, RPT)])
        if with_cnt:
            pltpu.sync_copy(cnt_sh.at[pl.ds(s * RPT, RPT)],
                            cnt_out.at[c, pl.ds(s * RPT, RPT)])


def _sc_agg(h, src, dst, with_cnt):
    mesh = plsc.VectorSubcoreMesh(core_axis_name="c", subcore_axis_name="s")
    out_type = [jax.ShapeDtypeStruct((NC, NP, HID), _f32)]
    scratch = [
        pltpu.VMEM((NIB, CHUNK), jnp.int32),
        pltpu.VMEM((NIB, CHUNK), jnp.int32),
        pltpu.VMEM((NBUF, CHUNK, HID), _f32),
    ]
    if with_cnt:
        out_type.append(jax.ShapeDtypeStruct((NC, NP), _f32))
        scratch += [pltpu.VMEM((CHUNK,), _f32), pltpu.VMEM((RPT,), _f32)]
    scratch.append(pltpu.VMEM_SHARED((NP, HID), _f32))
    if with_cnt:
        scratch.append(pltpu.VMEM_SHARED((NP,), _f32))
    scratch += [pltpu.SemaphoreType.DMA] * (NIB + NBUF)
    kern = functools.partial(
        pl.kernel,
        out_type=out_type,
        mesh=mesh,
        scratch_types=scratch,
    )(functools.partial(_sc_agg_body, with_cnt))
    return kern(h, src, dst)


_ROWS = 1024
_GRID = NP // _ROWS


def _mm(a, w):
    return lax.dot_general(a, w, (((1,), (0,)), ((), ())),
                           precision=lax.Precision.HIGHEST,
                           preferred_element_type=_f32)


def _encode_body(x_ref, w_ref, b_ref, o_ref):
    i = pl.program_id(0)
    h = jnp.maximum(_mm(x_ref[...], w_ref[...]) + b_ref[...], 0.0)
    rows = i * _ROWS + lax.broadcasted_iota(jnp.int32, (_ROWS, 1), 0)
    o_ref[...] = jnp.where(rows < N, h, 0.0)


def _tc_encode(x_pad, w, b):
    return pl.pallas_call(
        _encode_body,
        grid=(_GRID,),
        in_specs=[
            pl.BlockSpec((_ROWS, 8), lambda i: (i, 0)),
            pl.BlockSpec((8, HID), lambda i: (0, 0)),
            pl.BlockSpec((1, HID), lambda i: (0, 0)),
        ],
        out_specs=pl.BlockSpec((_ROWS, HID), lambda i: (i, 0)),
        out_shape=jax.ShapeDtypeStruct((NP, HID), _f32),
    )(x_pad, w, b)


def _sage_body(agg_ref, cnt_ref, h_ref, wl_ref, b_ref, wr_ref, o_ref):
    i = pl.program_id(0)
    cnt = cnt_ref[0] + cnt_ref[1]
    recip = 1.0 / jnp.maximum(cnt, 1.0)
    mean = (agg_ref[0] + agg_ref[1]) * recip
    out = _mm(mean, wl_ref[...]) + b_ref[...] + _mm(h_ref[...], wr_ref[...])
    out = jnp.maximum(out, 0.0)
    rows = i * _ROWS + lax.broadcasted_iota(jnp.int32, (_ROWS, 1), 0)
    o_ref[...] = jnp.where(rows < N, out, 0.0)


def _tc_sage(agg, cnt3, h, wl, b, wr):
    return pl.pallas_call(
        _sage_body,
        grid=(_GRID,),
        in_specs=[
            pl.BlockSpec((NC, _ROWS, HID), lambda i: (0, i, 0)),
            pl.BlockSpec((NC, _ROWS, 1), lambda i: (0, i, 0)),
            pl.BlockSpec((_ROWS, HID), lambda i: (i, 0)),
            pl.BlockSpec((HID, HID), lambda i: (0, 0)),
            pl.BlockSpec((1, HID), lambda i: (0, 0)),
            pl.BlockSpec((HID, HID), lambda i: (0, 0)),
        ],
        out_specs=pl.BlockSpec((_ROWS, HID), lambda i: (i, 0)),
        out_shape=jax.ShapeDtypeStruct((NP, HID), _f32),
    )(agg, cnt3, h, wl, b, wr)


_OROWS = 1000
_OGRID = N // _OROWS


def _out_body(h_ref, w_ref, b_ref, mean_ref, ns_ref):
    i = pl.program_id(0)
    ns = _mm(h_ref[...], w_ref[...]) + b_ref[...]
    ns_ref[...] = ns

    @pl.when(i == 0)
    def _():
        mean_ref[...] = jnp.zeros_like(mean_ref)

    mean_ref[...] += jnp.sum(ns, axis=0, keepdims=True)

    @pl.when(i == _OGRID - 1)
    def _():
        mean_ref[...] = mean_ref[...] * (1.0 / N)


def _tc_out(h, w, b):
    return pl.pallas_call(
        _out_body,
        grid=(_OGRID,),
        in_specs=[
            pl.BlockSpec((_OROWS, HID), lambda i: (i, 0)),
            pl.BlockSpec((HID, HID), lambda i: (0, 0)),
            pl.BlockSpec((1, HID), lambda i: (0, 0)),
        ],
        out_specs=[
            pl.BlockSpec((1, HID), lambda i: (0, 0)),
            pl.BlockSpec((_OROWS, HID), lambda i: (i, 0)),
        ],
        out_shape=[
            jax.ShapeDtypeStruct((1, HID), _f32),
            jax.ShapeDtypeStruct((N, HID), _f32),
        ],
    )(h, w, b)


def kernel(pos, atomic_number, edge_index,
           W_in, b_in, W1_l, b1, W1_r, W2_l, b2, W2_r, W_out, b_out):
    z = atomic_number.astype(_f32)[:, None] / 10.0
    x = jnp.concatenate([z, pos], axis=1)
    x_pad = jnp.pad(x, ((0, NP - N), (0, 4)))
    w_in8 = jnp.pad(W_in, ((0, 0), (0, 4))).T

    pad_k = jnp.arange(EPAD - E, dtype=jnp.int32)
    src = jnp.concatenate([edge_index[0], pad_k % N])
    dst = jnp.concatenate([edge_index[1], N + pad_k % (NP - N)])

    h0 = _tc_encode(x_pad, w_in8, b_in.reshape(1, HID))

    agg1, cnt = _sc_agg(h0, src, dst, True)
    cnt3 = cnt.reshape(NC, NP, 1)
    h1 = _tc_sage(agg1, cnt3, h0, W1_l.T, b1.reshape(1, HID), W1_r.T)

    (agg2,) = _sc_agg(h1, src, dst, False)
    h2 = _tc_sage(agg2, cnt3, h1, W2_l.T, b2.reshape(1, HID), W2_r.T)

    mean, node_states = _tc_out(h2, W_out.T, b_out.reshape(1, HID))
    return mean.reshape(HID), node_states


# speedup vs baseline: 12.5269x; 1.0358x over previous
import functools

import jax
import jax.numpy as jnp
from jax import lax
from jax.experimental import pallas as pl
from jax.experimental.pallas import tpu as pltpu
from jax.experimental.pallas import tpu_sc as plsc

N = 10000
E = 320000
HID = 128

NC = 2
NS = 16
NW = NC * NS
NP = 10240
RPT = NP // NS
CHUNK = 128
CPW0 = 80
CPW1 = 80
EPAD = NS * (CPW0 + CPW1) * CHUNK

NBUF = 2
NIB = 4

_f32 = jnp.float32


def _sc_agg_body(with_cnt, h_hbm, src_hbm, dst_hbm, *refs):
    if with_cnt:
        (agg_out, cnt_out, sidx, didx, bufs, ones_v, zcnt_v,
         agg_sh, cnt_sh, *sems) = refs
    else:
        (agg_out, sidx, didx, bufs, agg_sh, *sems) = refs
    isems, gsems = sems[:NIB], sems[NIB:]
    c = lax.axis_index("c")
    s = lax.axis_index("s")
    cpw = jnp.where(c == 0, CPW0, CPW1)
    chunk0 = jnp.where(c == 0, s * CPW0, NS * CPW0 + s * CPW1)

    def idx_start(ci, jj):
        e0 = (chunk0 + ci) * CHUNK
        pltpu.async_copy(src_hbm.at[pl.ds(e0, CHUNK)], sidx.at[jj], isems[jj])
        pltpu.async_copy(dst_hbm.at[pl.ds(e0, CHUNK)], didx.at[jj], isems[jj])

    def idx_wait(ci, jj):
        e0 = (chunk0 + ci) * CHUNK
        pltpu.make_async_copy(
            src_hbm.at[pl.ds(e0, CHUNK)], sidx.at[jj], isems[jj]).wait()
        pltpu.make_async_copy(
            dst_hbm.at[pl.ds(e0, CHUNK)], didx.at[jj], isems[jj]).wait()

    def gather_start(jj, j):
        pltpu.async_copy(h_hbm.at[sidx.at[jj]], bufs.at[j], gsems[j])

    def gather_wait(jj, j):
        pltpu.make_async_copy(h_hbm.at[sidx.at[jj]], bufs.at[j], gsems[j]).wait()

    for jj in range(NIB):
        idx_start(jj, jj)

    zeros16 = jnp.zeros((16,), _f32)

    def zero_rows(i, carry):
        for j in range(HID // 16):
            bufs[0, i, pl.ds(j * 16, 16)] = zeros16
        return carry
    lax.fori_loop(0, CHUNK, zero_rows, 0)
    zrow = bufs.at[0]
    for r in range(RPT // CHUNK):
        pltpu.sync_copy(zrow, agg_sh.at[pl.ds(s * RPT + r * CHUNK, CHUNK)])

    if with_cnt:
        def zero_cnt(i, carry):
            zcnt_v[pl.ds(i * 16, 16)] = zeros16
            return carry
        lax.fori_loop(0, RPT // 16, zero_cnt, 0)
        for j in range(CHUNK // 16):
            ones_v[pl.ds(j * 16, 16)] = jnp.ones((16,), _f32)
        pltpu.sync_copy(zcnt_v, cnt_sh.at[pl.ds(s * RPT, RPT)])

    for j in range(NBUF):
        idx_wait(j, j)
        gather_start(j, j)

    plsc.subcore_barrier()

    def do_chunk(ci, j, jj, jjn, start_gather, start_idx):
        gather_wait(jj, j)
        pltpu.sync_copy(bufs.at[j], agg_sh.at[didx.at[jj]], add=True)
        if with_cnt:
            pltpu.sync_copy(ones_v, cnt_sh.at[didx.at[jj]], add=True)
        if start_gather:
            idx_wait(ci + NBUF, jjn)
            gather_start(jjn, j)
        if start_idx:
            idx_start(ci + NIB, jj)

    gm = cpw // NIB - 1

    def group_body(g, carry):
        base = g * NIB
        for u in range(NIB):
            ci = base + u
            do_chunk(ci, u % NBUF, u, (u + NBUF) % NIB, True, True)
        return carry
    lax.fori_loop(0, gm, group_body, 0)

    base = cpw - NIB
    for u in range(NIB):
        ci = base + u
        do_chunk(ci, u % NBUF, u, (u + NBUF) % NIB, u < NIB - NBUF, False)

    plsc.subcore_barrier()

    pltpu.sync_copy(agg_sh.at[pl.ds(s * RPT, RPT)],
                    agg_out.at[c, pl.ds(s * RPT, RPT)])
    if with_cnt:
        pltpu.sync_copy(cnt_sh.at[pl.ds(s * RPT, RPT)],
                        cnt_out.at[c, pl.ds(s * RPT, RPT)])


def _sc_agg(h, src, dst, with_cnt):
    mesh = plsc.VectorSubcoreMesh(core_axis_name="c", subcore_axis_name="s")
    out_type = [jax.ShapeDtypeStruct((NC, NP, HID), _f32)]
    scratch = [
        pltpu.VMEM((NIB, CHUNK), jnp.int32),
        pltpu.VMEM((NIB, CHUNK), jnp.int32),
        pltpu.VMEM((NBUF, CHUNK, HID), _f32),
    ]
    if with_cnt:
        out_type.append(jax.ShapeDtypeStruct((NC, NP), _f32))
        scratch += [pltpu.VMEM((CHUNK,), _f32), pltpu.VMEM((RPT,), _f32)]
    scratch.append(pltpu.VMEM_SHARED((NP, HID), _f32))
    if with_cnt:
        scratch.append(pltpu.VMEM_SHARED((NP,), _f32))
    scratch += [pltpu.SemaphoreType.DMA] * (NIB + NBUF)
    kern = functools.partial(
        pl.kernel,
        out_type=out_type,
        mesh=mesh,
        scratch_types=scratch,
    )(functools.partial(_sc_agg_body, with_cnt))
    return kern(h, src, dst)


_ROWS = 1024
_GRID = NP // _ROWS


def _mm(a, w):
    return lax.dot_general(a, w, (((1,), (0,)), ((), ())),
                           precision=lax.Precision.DEFAULT,
                           preferred_element_type=_f32)


def _encode_body(x_ref, w_ref, b_ref, o_ref):
    i = pl.program_id(0)
    h = jnp.maximum(_mm(x_ref[...], w_ref[...]) + b_ref[...], 0.0)
    rows = i * _ROWS + lax.broadcasted_iota(jnp.int32, (_ROWS, 1), 0)
    o_ref[...] = jnp.where(rows < N, h, 0.0)


def _tc_encode(x_pad, w, b):
    return pl.pallas_call(
        _encode_body,
        grid=(_GRID,),
        in_specs=[
            pl.BlockSpec((_ROWS, 8), lambda i: (i, 0)),
            pl.BlockSpec((8, HID), lambda i: (0, 0)),
            pl.BlockSpec((1, HID), lambda i: (0, 0)),
        ],
        out_specs=pl.BlockSpec((_ROWS, HID), lambda i: (i, 0)),
        out_shape=jax.ShapeDtypeStruct((NP, HID), _f32),
    )(x_pad, w, b)


def _root_body(h_ref, wr_ref, o_ref):
    o_ref[...] = _mm(h_ref[...], wr_ref[...])


def _tc_root(h, wr):
    return pl.pallas_call(
        _root_body,
        grid=(_GRID,),
        in_specs=[
            pl.BlockSpec((_ROWS, HID), lambda i: (i, 0)),
            pl.BlockSpec((HID, HID), lambda i: (0, 0)),
        ],
        out_specs=pl.BlockSpec((_ROWS, HID), lambda i: (i, 0)),
        out_shape=jax.ShapeDtypeStruct((NP, HID), _f32),
    )(h, wr)


def _sage_body(agg_ref, cnt_ref, r_ref, wl_ref, b_ref, o_ref):
    i = pl.program_id(0)
    cnt = cnt_ref[0] + cnt_ref[1]
    recip = 1.0 / jnp.maximum(cnt, 1.0)
    mean = (agg_ref[0] + agg_ref[1]) * recip
    out = _mm(mean, wl_ref[...]) + b_ref[...] + r_ref[...]
    out = jnp.maximum(out, 0.0)
    rows = i * _ROWS + lax.broadcasted_iota(jnp.int32, (_ROWS, 1), 0)
    o_ref[...] = jnp.where(rows < N, out, 0.0)


def _tc_sage(agg, cnt3, r, wl, b):
    return pl.pallas_call(
        _sage_body,
        grid=(_GRID,),
        in_specs=[
            pl.BlockSpec((NC, _ROWS, HID), lambda i: (0, i, 0)),
            pl.BlockSpec((NC, _ROWS, 1), lambda i: (0, i, 0)),
            pl.BlockSpec((_ROWS, HID), lambda i: (i, 0)),
            pl.BlockSpec((HID, HID), lambda i: (0, 0)),
            pl.BlockSpec((1, HID), lambda i: (0, 0)),
        ],
        out_specs=pl.BlockSpec((_ROWS, HID), lambda i: (i, 0)),
        out_shape=jax.ShapeDtypeStruct((NP, HID), _f32),
    )(agg, cnt3, r, wl, b)


_OROWS = 1000
_OGRID = N // _OROWS


def _out_body(h_ref, w_ref, b_ref, mean_ref, ns_ref):
    i = pl.program_id(0)
    ns = _mm(h_ref[...], w_ref[...]) + b_ref[...]
    ns_ref[...] = ns

    @pl.when(i == 0)
    def _():
        mean_ref[...] = jnp.zeros_like(mean_ref)

    mean_ref[...] += jnp.sum(ns, axis=0, keepdims=True)

    @pl.when(i == _OGRID - 1)
    def _():
        mean_ref[...] = mean_ref[...] * (1.0 / N)


def _tc_out(h, w, b):
    return pl.pallas_call(
        _out_body,
        grid=(_OGRID,),
        in_specs=[
            pl.BlockSpec((_OROWS, HID), lambda i: (i, 0)),
            pl.BlockSpec((HID, HID), lambda i: (0, 0)),
            pl.BlockSpec((1, HID), lambda i: (0, 0)),
        ],
        out_specs=[
            pl.BlockSpec((1, HID), lambda i: (0, 0)),
            pl.BlockSpec((_OROWS, HID), lambda i: (i, 0)),
        ],
        out_shape=[
            jax.ShapeDtypeStruct((1, HID), _f32),
            jax.ShapeDtypeStruct((N, HID), _f32),
        ],
    )(h, w, b)


def kernel(pos, atomic_number, edge_index,
           W_in, b_in, W1_l, b1, W1_r, W2_l, b2, W2_r, W_out, b_out):
    z = atomic_number.astype(_f32)[:, None] / 10.0
    x = jnp.concatenate([z, pos], axis=1)
    x_pad = jnp.pad(x, ((0, NP - N), (0, 4)))
    w_in8 = jnp.pad(W_in, ((0, 0), (0, 4))).T

    pad_k = jnp.arange(EPAD - E, dtype=jnp.int32)
    src = jnp.concatenate([edge_index[0], pad_k % N])
    dst = jnp.concatenate([edge_index[1], N + pad_k % (NP - N)])

    h0 = _tc_encode(x_pad, w_in8, b_in.reshape(1, HID))

    agg1, cnt = _sc_agg(h0, src, dst, True)
    r1 = _tc_root(h0, W1_r.T)
    cnt3 = cnt.reshape(NC, NP, 1)
    h1 = _tc_sage(agg1, cnt3, r1, W1_l.T, b1.reshape(1, HID))

    (agg2,) = _sc_agg(h1, src, dst, False)
    r2 = _tc_root(h1, W2_r.T)
    h2 = _tc_sage(agg2, cnt3, r2, W2_l.T, b2.reshape(1, HID))

    mean, node_states = _tc_out(h2, W_out.T, b_out.reshape(1, HID))
    return mean.reshape(HID), node_states


# speedup vs baseline: 12.9942x; 1.0373x over previous
import functools

import jax
import jax.numpy as jnp
from jax import lax
from jax.experimental import pallas as pl
from jax.experimental.pallas import tpu as pltpu
from jax.experimental.pallas import tpu_sc as plsc

N = 10000
E = 320000
HID = 128

NC = 2
NS = 16
NW = NC * NS
NP = 10240
RPT = NP // NS
CHUNK = 128
CPW = 80
LAST_CPW = E // CHUNK - (NW - 1) * CPW

NBUF = 2
NIB = 4

_f32 = jnp.float32


def _sc_agg_body(with_cnt, h_hbm, src_hbm, dst_hbm, *refs):
    if with_cnt:
        (agg_out, cnt_out, sidx, didx, bufs, ones_v, zcnt_v,
         agg_sh, cnt_sh, *sems) = refs
    else:
        (agg_out, sidx, didx, bufs, agg_sh, *sems) = refs
    isems, gsems = sems[:NIB], sems[NIB:]
    c = lax.axis_index("c")
    s = lax.axis_index("s")
    w = c * NS + s
    cpw = jnp.where(w == NW - 1, LAST_CPW, CPW)
    chunk0 = w * CPW

    def idx_start(ci, jj):
        e0 = (chunk0 + ci) * CHUNK
        pltpu.async_copy(src_hbm.at[pl.ds(e0, CHUNK)], sidx.at[jj], isems[jj])
        pltpu.async_copy(dst_hbm.at[pl.ds(e0, CHUNK)], didx.at[jj], isems[jj])

    def idx_wait(ci, jj):
        e0 = (chunk0 + ci) * CHUNK
        pltpu.make_async_copy(
            src_hbm.at[pl.ds(e0, CHUNK)], sidx.at[jj], isems[jj]).wait()
        pltpu.make_async_copy(
            dst_hbm.at[pl.ds(e0, CHUNK)], didx.at[jj], isems[jj]).wait()

    def gather_start(jj, j):
        pltpu.async_copy(h_hbm.at[sidx.at[jj]], bufs.at[j], gsems[j])

    def gather_wait(jj, j):
        pltpu.make_async_copy(h_hbm.at[sidx.at[jj]], bufs.at[j], gsems[j]).wait()

    for jj in range(NIB):
        idx_start(jj, jj)

    zeros16 = jnp.zeros((16,), _f32)

    def zero_rows(i, carry):
        for j in range(HID // 16):
            bufs[0, i, pl.ds(j * 16, 16)] = zeros16
        return carry
    lax.fori_loop(0, CHUNK, zero_rows, 0)
    zrow = bufs.at[0]
    for r in range(RPT // CHUNK):
        pltpu.sync_copy(zrow, agg_sh.at[pl.ds(s * RPT + r * CHUNK, CHUNK)])

    if with_cnt:
        def zero_cnt(i, carry):
            zcnt_v[pl.ds(i * 16, 16)] = zeros16
            return carry
        lax.fori_loop(0, RPT // 16, zero_cnt, 0)
        for j in range(CHUNK // 16):
            ones_v[pl.ds(j * 16, 16)] = jnp.ones((16,), _f32)
        pltpu.sync_copy(zcnt_v, cnt_sh.at[pl.ds(s * RPT, RPT)])

    for j in range(NBUF):
        idx_wait(j, j)
        gather_start(j, j)

    plsc.subcore_barrier()

    def do_chunk(ci, j, jj, jjn, start_gather, start_idx):
        gather_wait(jj, j)
        pltpu.sync_copy(bufs.at[j], agg_sh.at[didx.at[jj]], add=True)
        if with_cnt:
            pltpu.sync_copy(ones_v, cnt_sh.at[didx.at[jj]], add=True)
        if start_gather:
            idx_wait(ci + NBUF, jjn)
            gather_start(jjn, j)
        if start_idx:
            idx_start(ci + NIB, jj)

    gm = cpw // NIB - 1

    def group_body(g, carry):
        base = g * NIB
        for u in range(NIB):
            ci = base + u
            do_chunk(ci, u % NBUF, u, (u + NBUF) % NIB, True, True)
        return carry
    lax.fori_loop(0, gm, group_body, 0)

    base = cpw - NIB
    for u in range(NIB):
        ci = base + u
        do_chunk(ci, u % NBUF, u, (u + NBUF) % NIB, u < NIB - NBUF, False)

    plsc.subcore_barrier()

    pltpu.sync_copy(agg_sh.at[pl.ds(s * RPT, RPT)],
                    agg_out.at[c, pl.ds(s * RPT, RPT)])
    if with_cnt:
        pltpu.sync_copy(cnt_sh.at[pl.ds(s * RPT, RPT)],
                        cnt_out.at[c, pl.ds(s * RPT, RPT)])


def _sc_agg(h, src, dst, with_cnt):
    mesh = plsc.VectorSubcoreMesh(core_axis_name="c", subcore_axis_name="s")
    out_type = [jax.ShapeDtypeStruct((NC, NP, HID), _f32)]
    scratch = [
        pltpu.VMEM((NIB, CHUNK), jnp.int32),
        pltpu.VMEM((NIB, CHUNK), jnp.int32),
        pltpu.VMEM((NBUF, CHUNK, HID), _f32),
    ]
    if with_cnt:
        out_type.append(jax.ShapeDtypeStruct((NC, NP), _f32))
        scratch += [pltpu.VMEM((CHUNK,), _f32), pltpu.VMEM((RPT,), _f32)]
    scratch.append(pltpu.VMEM_SHARED((NP, HID), _f32))
    if with_cnt:
        scratch.append(pltpu.VMEM_SHARED((NP,), _f32))
    scratch += [pltpu.SemaphoreType.DMA] * (NIB + NBUF)
    kern = functools.partial(
        pl.kernel,
        out_type=out_type,
        mesh=mesh,
        scratch_types=scratch,
    )(functools.partial(_sc_agg_body, with_cnt))
    return kern(h, src, dst)


_ROWS = 1024
_GRID = NP // _ROWS


def _mm(a, w):
    return lax.dot_general(a, w, (((1,), (0,)), ((), ())),
                           precision=lax.Precision.DEFAULT,
                           preferred_element_type=_f32)


def _encode_body(x_ref, w_ref, b_ref, o_ref):
    i = pl.program_id(0)
    h = jnp.maximum(_mm(x_ref[...], w_ref[...]) + b_ref[...], 0.0)
    rows = i * _ROWS + lax.broadcasted_iota(jnp.int32, (_ROWS, 1), 0)
    o_ref[...] = jnp.where(rows < N, h, 0.0)


def _tc_encode(x_pad, w, b):
    return pl.pallas_call(
        _encode_body,
        grid=(_GRID,),
        in_specs=[
            pl.BlockSpec((_ROWS, 8), lambda i: (i, 0)),
            pl.BlockSpec((8, HID), lambda i: (0, 0)),
            pl.BlockSpec((1, HID), lambda i: (0, 0)),
        ],
        out_specs=pl.BlockSpec((_ROWS, HID), lambda i: (i, 0)),
        out_shape=jax.ShapeDtypeStruct((NP, HID), _f32),
    )(x_pad, w, b)


def _root_body(h_ref, wr_ref, o_ref):
    o_ref[...] = _mm(h_ref[...], wr_ref[...])


def _tc_root(h, wr):
    return pl.pallas_call(
        _root_body,
        grid=(_GRID,),
        in_specs=[
            pl.BlockSpec((_ROWS, HID), lambda i: (i, 0)),
            pl.BlockSpec((HID, HID), lambda i: (0, 0)),
        ],
        out_specs=pl.BlockSpec((_ROWS, HID), lambda i: (i, 0)),
        out_shape=jax.ShapeDtypeStruct((NP, HID), _f32),
    )(h, wr)


def _sage_body(agg_ref, cnt_ref, r_ref, wl_ref, b_ref, o_ref):
    i = pl.program_id(0)
    cnt = cnt_ref[0] + cnt_ref[1]
    recip = 1.0 / jnp.maximum(cnt, 1.0)
    mean = (agg_ref[0] + agg_ref[1]) * recip
    out = _mm(mean, wl_ref[...]) + b_ref[...] + r_ref[...]
    out = jnp.maximum(out, 0.0)
    rows = i * _ROWS + lax.broadcasted_iota(jnp.int32, (_ROWS, 1), 0)
    o_ref[...] = jnp.where(rows < N, out, 0.0)


def _tc_sage(agg, cnt3, r, wl, b):
    return pl.pallas_call(
        _sage_body,
        grid=(_GRID,),
        in_specs=[
            pl.BlockSpec((NC, _ROWS, HID), lambda i: (0, i, 0)),
            pl.BlockSpec((NC, _ROWS, 1), lambda i: (0, i, 0)),
            pl.BlockSpec((_ROWS, HID), lambda i: (i, 0)),
            pl.BlockSpec((HID, HID), lambda i: (0, 0)),
            pl.BlockSpec((1, HID), lambda i: (0, 0)),
        ],
        out_specs=pl.BlockSpec((_ROWS, HID), lambda i: (i, 0)),
        out_shape=jax.ShapeDtypeStruct((NP, HID), _f32),
    )(agg, cnt3, r, wl, b)


_OROWS = 1000
_OGRID = N // _OROWS


def _sage_out_body(agg_ref, cnt_ref, r_ref, wl_ref, b_ref, wo_ref, bo_ref,
                   mean_ref, ns_ref):
    i = pl.program_id(0)
    cnt = cnt_ref[0] + cnt_ref[1]
    recip = 1.0 / jnp.maximum(cnt, 1.0)
    mean = (agg_ref[0] + agg_ref[1]) * recip
    h2 = jnp.maximum(_mm(mean, wl_ref[...]) + b_ref[...] + r_ref[...], 0.0)
    ns = _mm(h2, wo_ref[...]) + bo_ref[...]
    ns_ref[...] = ns

    @pl.when(i == 0)
    def _():
        mean_ref[...] = jnp.zeros_like(mean_ref)

    mean_ref[...] += jnp.sum(ns, axis=0, keepdims=True)

    @pl.when(i == _OGRID - 1)
    def _():
        mean_ref[...] = mean_ref[...] * (1.0 / N)


def _tc_sage_out(agg, cnt3, r, wl, b, wo, bo):
    return pl.pallas_call(
        _sage_out_body,
        grid=(_OGRID,),
        in_specs=[
            pl.BlockSpec((NC, _OROWS, HID), lambda i: (0, i, 0)),
            pl.BlockSpec((NC, _OROWS, 1), lambda i: (0, i, 0)),
            pl.BlockSpec((_OROWS, HID), lambda i: (i, 0)),
            pl.BlockSpec((HID, HID), lambda i: (0, 0)),
            pl.BlockSpec((1, HID), lambda i: (0, 0)),
            pl.BlockSpec((HID, HID), lambda i: (0, 0)),
            pl.BlockSpec((1, HID), lambda i: (0, 0)),
        ],
        out_specs=[
            pl.BlockSpec((1, HID), lambda i: (0, 0)),
            pl.BlockSpec((_OROWS, HID), lambda i: (i, 0)),
        ],
        out_shape=[
            jax.ShapeDtypeStruct((1, HID), _f32),
            jax.ShapeDtypeStruct((N, HID), _f32),
        ],
    )(agg, cnt3, r, wl, b, wo, bo)


def kernel(pos, atomic_number, edge_index,
           W_in, b_in, W1_l, b1, W1_r, W2_l, b2, W2_r, W_out, b_out):
    z = atomic_number.astype(_f32)[:, None] / 10.0
    x = jnp.concatenate([z, pos], axis=1)
    x_pad = jnp.pad(x, ((0, NP - N), (0, 4)))
    w_in8 = jnp.pad(W_in, ((0, 0), (0, 4))).T

    src = edge_index[0]
    dst = edge_index[1]

    h0 = _tc_encode(x_pad, w_in8, b_in.reshape(1, HID))

    agg1, cnt = _sc_agg(h0, src, dst, True)
    r1 = _tc_root(h0, W1_r.T)
    cnt3 = cnt.reshape(NC, NP, 1)
    h1 = _tc_sage(agg1, cnt3, r1, W1_l.T, b1.reshape(1, HID))

    (agg2,) = _sc_agg(h1, src, dst, False)
    r2 = _tc_root(h1, W2_r.T)

    mean, node_states = _tc_sage_out(
        agg2, cnt3, r2, W2_l.T, b2.reshape(1, HID),
        W_out.T, b_out.reshape(1, HID))
    return mean.reshape(HID), node_states


# speedup vs baseline: 13.0308x; 1.0028x over previous
import functools

import jax
import jax.numpy as jnp
from jax import lax
from jax.experimental import pallas as pl
from jax.experimental.pallas import tpu as pltpu
from jax.experimental.pallas import tpu_sc as plsc

N = 10000
E = 320000
HID = 128

NC = 2
NS = 16
NW = NC * NS
NP = 10240
RPT = NP // NS
CHUNK = 128
CPW = 80
LAST_CPW = E // CHUNK - (NW - 1) * CPW

NBUF = 2
NIB = 4

_f32 = jnp.float32


def _sc_agg_body(with_cnt, h_hbm, src_hbm, dst_hbm, *refs):
    if with_cnt:
        (agg_out, cnt_out, sidx, didx, bufs, ones_v, zcnt_v,
         agg_sh, cnt_sh, *sems) = refs
    else:
        (agg_out, sidx, didx, bufs, agg_sh, *sems) = refs
    isems, gsems, ssems = sems[:NIB], sems[NIB:NIB + NBUF], sems[NIB + NBUF:]
    c = lax.axis_index("c")
    s = lax.axis_index("s")
    w = c * NS + s
    cpw = jnp.where(w == NW - 1, LAST_CPW, CPW)
    chunk0 = w * CPW

    def idx_start(ci, jj):
        e0 = (chunk0 + ci) * CHUNK
        pltpu.async_copy(src_hbm.at[pl.ds(e0, CHUNK)], sidx.at[jj], isems[jj])
        pltpu.async_copy(dst_hbm.at[pl.ds(e0, CHUNK)], didx.at[jj], isems[jj])

    def idx_wait(ci, jj):
        e0 = (chunk0 + ci) * CHUNK
        pltpu.make_async_copy(
            src_hbm.at[pl.ds(e0, CHUNK)], sidx.at[jj], isems[jj]).wait()
        pltpu.make_async_copy(
            dst_hbm.at[pl.ds(e0, CHUNK)], didx.at[jj], isems[jj]).wait()

    def gather_start(jj, j):
        pltpu.async_copy(h_hbm.at[sidx.at[jj]], bufs.at[j], gsems[j])

    def gather_wait(jj, j):
        pltpu.make_async_copy(h_hbm.at[sidx.at[jj]], bufs.at[j], gsems[j]).wait()

    for jj in range(NIB):
        idx_start(jj, jj)

    zeros16 = jnp.zeros((16,), _f32)

    def zero_rows(i, carry):
        for j in range(HID // 16):
            bufs[0, i, pl.ds(j * 16, 16)] = zeros16
        return carry
    lax.fori_loop(0, CHUNK, zero_rows, 0)
    zrow = bufs.at[0]
    for r in range(RPT // CHUNK):
        pltpu.sync_copy(zrow, agg_sh.at[pl.ds(s * RPT + r * CHUNK, CHUNK)])

    if with_cnt:
        def zero_cnt(i, carry):
            zcnt_v[pl.ds(i * 16, 16)] = zeros16
            return carry
        lax.fori_loop(0, RPT // 16, zero_cnt, 0)
        for j in range(CHUNK // 16):
            ones_v[pl.ds(j * 16, 16)] = jnp.ones((16,), _f32)
        pltpu.sync_copy(zcnt_v, cnt_sh.at[pl.ds(s * RPT, RPT)])

    for j in range(NBUF):
        idx_wait(j, j)
        gather_start(j, j)

    plsc.subcore_barrier()

    def do_chunk(ci, j, jj, jjn, start_gather, start_idx):
        gather_wait(jj, j)
        scat = pltpu.async_copy(
            bufs.at[j], agg_sh.at[didx.at[jj]], ssems[j], add=True)
        if with_cnt:
            pltpu.sync_copy(ones_v, cnt_sh.at[didx.at[jj]], add=True)
        if start_gather:
            idx_wait(ci + NBUF, jjn)
        scat.wait()
        if start_gather:
            gather_start(jjn, j)
        if start_idx:
            idx_start(ci + NIB, jj)

    gm = cpw // NIB - 1

    def group_body(g, carry):
        base = g * NIB
        for u in range(NIB):
            ci = base + u
            do_chunk(ci, u % NBUF, u, (u + NBUF) % NIB, True, True)
        return carry
    lax.fori_loop(0, gm, group_body, 0)

    base = cpw - NIB
    for u in range(NIB):
        ci = base + u
        do_chunk(ci, u % NBUF, u, (u + NBUF) % NIB, u < NIB - NBUF, False)

    plsc.subcore_barrier()

    pltpu.sync_copy(agg_sh.at[pl.ds(s * RPT, RPT)],
                    agg_out.at[c, pl.ds(s * RPT, RPT)])
    if with_cnt:
        pltpu.sync_copy(cnt_sh.at[pl.ds(s * RPT, RPT)],
                        cnt_out.at[c, pl.ds(s * RPT, RPT)])


def _sc_agg(h, src, dst, with_cnt):
    mesh = plsc.VectorSubcoreMesh(core_axis_name="c", subcore_axis_name="s")
    out_type = [jax.ShapeDtypeStruct((NC, NP, HID), _f32)]
    scratch = [
        pltpu.VMEM((NIB, CHUNK), jnp.int32),
        pltpu.VMEM((NIB, CHUNK), jnp.int32),
        pltpu.VMEM((NBUF, CHUNK, HID), _f32),
    ]
    if with_cnt:
        out_type.append(jax.ShapeDtypeStruct((NC, NP), _f32))
        scratch += [pltpu.VMEM((CHUNK,), _f32), pltpu.VMEM((RPT,), _f32)]
    scratch.append(pltpu.VMEM_SHARED((NP, HID), _f32))
    if with_cnt:
        scratch.append(pltpu.VMEM_SHARED((NP,), _f32))
    scratch += [pltpu.SemaphoreType.DMA] * (NIB + 2 * NBUF)
    kern = functools.partial(
        pl.kernel,
        out_type=out_type,
        mesh=mesh,
        scratch_types=scratch,
    )(functools.partial(_sc_agg_body, with_cnt))
    return kern(h, src, dst)


_ROWS = 1024
_GRID = NP // _ROWS


def _mm(a, w):
    return lax.dot_general(a, w, (((1,), (0,)), ((), ())),
                           precision=lax.Precision.DEFAULT,
                           preferred_element_type=_f32)


def _encode_body(x_ref, w_ref, b_ref, o_ref):
    i = pl.program_id(0)
    h = jnp.maximum(_mm(x_ref[...], w_ref[...]) + b_ref[...], 0.0)
    rows = i * _ROWS + lax.broadcasted_iota(jnp.int32, (_ROWS, 1), 0)
    o_ref[...] = jnp.where(rows < N, h, 0.0)


def _tc_encode(x_pad, w, b):
    return pl.pallas_call(
        _encode_body,
        grid=(_GRID,),
        in_specs=[
            pl.BlockSpec((_ROWS, 8), lambda i: (i, 0)),
            pl.BlockSpec((8, HID), lambda i: (0, 0)),
            pl.BlockSpec((1, HID), lambda i: (0, 0)),
        ],
        out_specs=pl.BlockSpec((_ROWS, HID), lambda i: (i, 0)),
        out_shape=jax.ShapeDtypeStruct((NP, HID), _f32),
    )(x_pad, w, b)


def _root_body(h_ref, wr_ref, o_ref):
    o_ref[...] = _mm(h_ref[...], wr_ref[...])


def _tc_root(h, wr):
    return pl.pallas_call(
        _root_body,
        grid=(_GRID,),
        in_specs=[
            pl.BlockSpec((_ROWS, HID), lambda i: (i, 0)),
            pl.BlockSpec((HID, HID), lambda i: (0, 0)),
        ],
        out_specs=pl.BlockSpec((_ROWS, HID), lambda i: (i, 0)),
        out_shape=jax.ShapeDtypeStruct((NP, HID), _f32),
    )(h, wr)


def _sage_body(agg_ref, cnt_ref, r_ref, wl_ref, b_ref, o_ref):
    i = pl.program_id(0)
    cnt = cnt_ref[0] + cnt_ref[1]
    recip = 1.0 / jnp.maximum(cnt, 1.0)
    mean = (agg_ref[0] + agg_ref[1]) * recip
    out = _mm(mean, wl_ref[...]) + b_ref[...] + r_ref[...]
    out = jnp.maximum(out, 0.0)
    rows = i * _ROWS + lax.broadcasted_iota(jnp.int32, (_ROWS, 1), 0)
    o_ref[...] = jnp.where(rows < N, out, 0.0)


def _tc_sage(agg, cnt3, r, wl, b):
    return pl.pallas_call(
        _sage_body,
        grid=(_GRID,),
        in_specs=[
            pl.BlockSpec((NC, _ROWS, HID), lambda i: (0, i, 0)),
            pl.BlockSpec((NC, _ROWS, 1), lambda i: (0, i, 0)),
            pl.BlockSpec((_ROWS, HID), lambda i: (i, 0)),
            pl.BlockSpec((HID, HID), lambda i: (0, 0)),
            pl.BlockSpec((1, HID), lambda i: (0, 0)),
        ],
        out_specs=pl.BlockSpec((_ROWS, HID), lambda i: (i, 0)),
        out_shape=jax.ShapeDtypeStruct((NP, HID), _f32),
    )(agg, cnt3, r, wl, b)


_OROWS = 1000
_OGRID = N // _OROWS


def _sage_out_body(agg_ref, cnt_ref, r_ref, wl_ref, b_ref, wo_ref, bo_ref,
                   mean_ref, ns_ref):
    i = pl.program_id(0)
    cnt = cnt_ref[0] + cnt_ref[1]
    recip = 1.0 / jnp.maximum(cnt, 1.0)
    mean = (agg_ref[0] + agg_ref[1]) * recip
    h2 = jnp.maximum(_mm(mean, wl_ref[...]) + b_ref[...] + r_ref[...], 0.0)
    ns = _mm(h2, wo_ref[...]) + bo_ref[...]
    ns_ref[...] = ns

    @pl.when(i == 0)
    def _():
        mean_ref[...] = jnp.zeros_like(mean_ref)

    mean_ref[...] += jnp.sum(ns, axis=0, keepdims=True)

    @pl.when(i == _OGRID - 1)
    def _():
        mean_ref[...] = mean_ref[...] * (1.0 / N)


def _tc_sage_out(agg, cnt3, r, wl, b, wo, bo):
    return pl.pallas_call(
        _sage_out_body,
        grid=(_OGRID,),
        in_specs=[
            pl.BlockSpec((NC, _OROWS, HID), lambda i: (0, i, 0)),
            pl.BlockSpec((NC, _OROWS, 1), lambda i: (0, i, 0)),
            pl.BlockSpec((_OROWS, HID), lambda i: (i, 0)),
            pl.BlockSpec((HID, HID), lambda i: (0, 0)),
            pl.BlockSpec((1, HID), lambda i: (0, 0)),
            pl.BlockSpec((HID, HID), lambda i: (0, 0)),
            pl.BlockSpec((1, HID), lambda i: (0, 0)),
        ],
        out_specs=[
            pl.BlockSpec((1, HID), lambda i: (0, 0)),
            pl.BlockSpec((_OROWS, HID), lambda i: (i, 0)),
        ],
        out_shape=[
            jax.ShapeDtypeStruct((1, HID), _f32),
            jax.ShapeDtypeStruct((N, HID), _f32),
        ],
    )(agg, cnt3, r, wl, b, wo, bo)


def kernel(pos, atomic_number, edge_index,
           W_in, b_in, W1_l, b1, W1_r, W2_l, b2, W2_r, W_out, b_out):
    z = atomic_number.astype(_f32)[:, None] / 10.0
    x = jnp.concatenate([z, pos], axis=1)
    x_pad = jnp.pad(x, ((0, NP - N), (0, 4)))
    w_in8 = jnp.pad(W_in, ((0, 0), (0, 4))).T

    src = edge_index[0]
    dst = edge_index[1]

    h0 = _tc_encode(x_pad, w_in8, b_in.reshape(1, HID))

    agg1, cnt = _sc_agg(h0, src, dst, True)
    r1 = _tc_root(h0, W1_r.T)
    cnt3 = cnt.reshape(NC, NP, 1)
    h1 = _tc_sage(agg1, cnt3, r1, W1_l.T, b1.reshape(1, HID))

    (agg2,) = _sc_agg(h1, src, dst, False)
    r2 = _tc_root(h1, W2_r.T)

    mean, node_states = _tc_sage_out(
        agg2, cnt3, r2, W2_l.T, b2.reshape(1, HID),
        W_out.T, b_out.reshape(1, HID))
    return mean.reshape(HID), node_states
